# baseline (speedup 1.0000x reference)
import functools

import jax
import jax.numpy as jnp
from jax import lax
from jax.experimental import pallas as pl
from jax.experimental.pallas import tpu as pltpu
from jax.experimental.pallas import tpu_sc as plsc


def _make_sc_kernel(V, D, B, H, chunk_b, idx_row):
    info = plsc.get_sparse_core_info()
    NC, NS, L = info.num_cores, info.num_subcores, info.num_lanes
    NW = NC * NS
    N = B * H
    chunk_rows = chunk_b * H
    assert D % L == 0 and B % (NW * chunk_b) == 0 and chunk_rows % idx_row == 0
    assert chunk_rows % L == 0 and (N // NW) % 8 == 0
    n_chunks = B // (NW * chunk_b)
    g_streams = chunk_rows // idx_row
    n_groups = chunk_rows // L
    rows_per_worker = N // NW

    mesh = plsc.VectorSubcoreMesh(core_axis_name="c", subcore_axis_name="s")

    @functools.partial(
        pl.kernel,
        mesh=mesh,
        out_type=jax.ShapeDtypeStruct((B, H, D), jnp.float32),
        scratch_types=[
            pltpu.VMEM((rows_per_worker,), jnp.int32),
            pltpu.VMEM((chunk_rows, D), jnp.float32),
            pltpu.VMEM((chunk_rows, D), jnp.float32),
            pltpu.SemaphoreType.DMA,
            pltpu.SemaphoreType.DMA,
            pltpu.SemaphoreType.DMA,
            pltpu.SemaphoreType.DMA,
        ],
        compiler_params=pltpu.CompilerParams(
            needs_layout_passes=False, use_tc_tiling_on_sc=False
        ),
    )
    def sc_kernel(emb_hbm, idx_hbm, out_hbm, idx_v, rows_a, rows_b,
                  gsem_a, gsem_b, osem_a, osem_b):
        wid = lax.axis_index("s") * NC + lax.axis_index("c")
        lane = lax.iota(jnp.int32, L)
        zero16 = jnp.zeros((L,), jnp.int32)
        pltpu.sync_copy(idx_hbm.at[pl.ds(wid * rows_per_worker, rows_per_worker)],
                        idx_v)

        bufs = (rows_a, rows_b)
        gsems = (gsem_a, gsem_b)
        osems = (osem_a, osem_b)

        def fire_gathers(k, buf, gsem):
            return [
                pltpu.async_copy(
                    emb_hbm.at[idx_v.at[pl.ds(k * chunk_rows + j * idx_row,
                                              idx_row)]],
                    buf.at[pl.ds(j * idx_row, idx_row)],
                    gsem,
                )
                for j in range(g_streams)
            ]

        def fire_writeout(k, buf, osem):
            b0 = wid * n_chunks * chunk_b + k * chunk_b
            return [
                pltpu.async_copy(
                    buf.at[pl.ds(b * H, H)],
                    out_hbm.at[b0 + b],
                    osem,
                )
                for b in range(chunk_b)
            ]

        def fixup_chunk(buf):
            n_acc = 8
            mask_d = D - 1
            diags = [(lane + s) & mask_d for s in range(D)]

            @plsc.parallel_loop(0, n_groups)
            def grp_body(g):
                row_ids = g * L + lane
                accs = [jnp.zeros((L,), jnp.float32) for _ in range(n_acc)]
                for s in range(D):
                    v = plsc.load_gather(buf, [row_ids, diags[s]])
                    accs[s % n_acc] = accs[s % n_acc] + v * v
                v0 = plsc.load_gather(buf, [row_ids, zero16])
                while len(accs) > 1:
                    accs = [a + b for a, b in zip(accs[0::2], accs[1::2])]
                x = jnp.maximum(accs[0] - v0 * v0 + 1.0, 1.0 + 1e-12)
                i = lax.bitcast_convert_type(x, jnp.int32)
                i = 0x5F3759DF - lax.shift_right_arithmetic(i, 1)
                y = lax.bitcast_convert_type(i, jnp.float32)
                y = y * (1.5 - 0.5 * x * y * y)
                y = y * (1.5 - 0.5 * x * y * y)
                y = y * (1.5 - 0.5 * x * y * y)
                x0 = x * y
                plsc.store_scatter(buf, [row_ids, zero16], x0)

        gather_copies = {0: fire_gathers(0, bufs[0], gsems[0])}
        out_copies = {}
        for k in range(n_chunks):
            p = k % 2
            if k + 1 < n_chunks:
                if k - 1 >= 0:
                    for c in out_copies.pop(k - 1):
                        c.wait()
                gather_copies[k + 1] = fire_gathers(
                    k + 1, bufs[(k + 1) % 2], gsems[(k + 1) % 2]
                )
            for c in gather_copies.pop(k):
                c.wait()
            fixup_chunk(bufs[p])
            out_copies[k] = fire_writeout(k, bufs[p], osems[p])
        for k in list(out_copies):
            for c in out_copies.pop(k):
                c.wait()

    return sc_kernel


def kernel(embeddings, indices):
    V, D = embeddings.shape
    B, H = indices.shape
    idx_flat = indices.reshape(-1).astype(jnp.int32)
    sc = _make_sc_kernel(V, D, B, H, chunk_b=16, idx_row=80)
    return sc(embeddings, idx_flat)


# speedup vs baseline: 1.4793x; 1.4793x over previous
import functools

import jax
import jax.numpy as jnp
from jax import lax
from jax.experimental import pallas as pl
from jax.experimental.pallas import tpu as pltpu
from jax.experimental.pallas import tpu_sc as plsc


def _make_sc_kernel(V, D, B, H):
    info = plsc.get_sparse_core_info()
    NC, NS, L = info.num_cores, info.num_subcores, info.num_lanes
    NW = NC * NS
    N = B * H
    BL = B // NW
    assert D % L == 0 and B % NW == 0 and BL % L == 0 and H % 2 == 0
    n_grp = BL // L
    d_tiles = D // 8
    out_rows = N * D // 128

    mesh = plsc.VectorSubcoreMesh(core_axis_name="c", subcore_axis_name="s")

    @functools.partial(
        pl.kernel,
        mesh=mesh,
        out_type=jax.ShapeDtypeStruct((out_rows, 128), jnp.float32),
        scratch_types=[
            pltpu.VMEM((BL * H,), jnp.int32),
            pltpu.VMEM((BL,), jnp.int32),
            pltpu.VMEM((BL,), jnp.int32),
            pltpu.VMEM((BL, D), jnp.float32),
            pltpu.VMEM((BL, D), jnp.float32),
            pltpu.VMEM((D, BL), jnp.float32),
            pltpu.VMEM((D, BL), jnp.float32),
            pltpu.SemaphoreType.DMA,
            pltpu.SemaphoreType.DMA,
            pltpu.SemaphoreType.DMA,
            pltpu.SemaphoreType.DMA,
        ],
        compiler_params=pltpu.CompilerParams(
            needs_layout_passes=False, use_tc_tiling_on_sc=False
        ),
    )
    def sc_kernel(emb_hbm, idx_hbm, out_hbm, idx_v, icol_a, icol_b,
                  gbuf_a, gbuf_b, tbuf_a, tbuf_b,
                  gsem_a, gsem_b, osem_a, osem_b):
        wid = lax.axis_index("s") * NC + lax.axis_index("c")
        lane = lax.iota(jnp.int32, L)
        zero16 = jnp.zeros((L,), jnp.int32)
        mask_d = D - 1
        pltpu.sync_copy(idx_hbm.at[pl.ds(wid * BL * H, BL * H)], idx_v)

        def build_idxcol(h, icol):
            for gg in range(n_grp):
                bvec = (gg * L + lane) * H + h
                vals = plsc.load_gather(idx_v, [bvec])
                icol[pl.ds(gg * L, L)] = vals

        def fire_gather(icol, gbuf, gsem):
            return pltpu.async_copy(emb_hbm.at[icol], gbuf, gsem)

        def drain_gather(icol, gbuf, gsem):
            pltpu.make_async_copy(emb_hbm.at[icol], gbuf, gsem).wait()

        def fixup_transpose(gbuf, tbuf):
            @plsc.parallel_loop(0, n_grp)
            def grp_body(gg):
                b_ids = gg * L + lane

                def s_body(so, accs):
                    new = list(accs)
                    for si in range(8):
                        diag = (lane + (so * 8 + si)) & mask_d
                        v = plsc.load_gather(gbuf, [b_ids, diag])
                        new[si] = new[si] + v * v
                        plsc.store_scatter(tbuf, [diag, b_ids], v)
                    return tuple(new)

                accs = list(lax.fori_loop(
                    0, D // 8, s_body,
                    tuple(jnp.zeros((L,), jnp.float32) for _ in range(8)),
                ))
                v0 = plsc.load_gather(gbuf, [b_ids, zero16])
                while len(accs) > 1:
                    accs = [a + b for a, b in zip(accs[0::2], accs[1::2])]
                x = jnp.maximum(accs[0] - v0 * v0 + 1.0, 1.0 + 1e-12)
                i = lax.bitcast_convert_type(x, jnp.int32)
                i = 0x5F3759DF - lax.shift_right_arithmetic(i, 1)
                y = lax.bitcast_convert_type(i, jnp.float32)
                y = y * (1.5 - 0.5 * x * y * y)
                y = y * (1.5 - 0.5 * x * y * y)
                y = y * (1.5 - 0.5 * x * y * y)
                plsc.store_scatter(tbuf, [zero16, b_ids], x * y)

        def fire_writeout(h, tbuf, osem):
            return [
                pltpu.async_copy(
                    tbuf.at[pl.ds(tr * 8, 8)],
                    out_hbm.at[pl.ds(((h * d_tiles + tr) * NW + wid) * 8, 8)],
                    osem,
                )
                for tr in range(d_tiles)
            ]

        def drain_writeout(tbuf, osem):
            pltpu.make_async_copy(tbuf, out_hbm.at[pl.ds(0, D)], osem).wait()

        build_idxcol(0, icol_a)
        fire_gather(icol_a, gbuf_a, gsem_a)
        fire_writeout(0, tbuf_a, osem_a)
        fire_writeout(1, tbuf_b, osem_b)

        def round_body(q, carry):
            h0 = 2 * q
            h1 = h0 + 1
            build_idxcol(h1, icol_b)
            fire_gather(icol_b, gbuf_b, gsem_b)
            drain_gather(icol_a, gbuf_a, gsem_a)
            drain_writeout(tbuf_a, osem_a)
            fixup_transpose(gbuf_a, tbuf_a)
            fire_writeout(h0, tbuf_a, osem_a)
            h2 = jnp.minimum(h0 + 2, H - 2)
            build_idxcol(h2, icol_a)
            fire_gather(icol_a, gbuf_a, gsem_a)
            drain_gather(icol_b, gbuf_b, gsem_b)
            drain_writeout(tbuf_b, osem_b)
            fixup_transpose(gbuf_b, tbuf_b)
            fire_writeout(h1, tbuf_b, osem_b)
            return carry

        lax.fori_loop(0, H // 2, round_body, 0)
        drain_gather(icol_a, gbuf_a, gsem_a)
        drain_writeout(tbuf_a, osem_a)
        drain_writeout(tbuf_b, osem_b)

    return sc_kernel


def kernel(embeddings, indices):
    V, D = embeddings.shape
    B, H = indices.shape
    idx_flat = indices.reshape(-1).astype(jnp.int32)
    sc = _make_sc_kernel(V, D, B, H)
    out2 = sc(embeddings, idx_flat)
    nw = 32
    out5 = out2.reshape(H, D // 8, nw, 8, B // nw)
    return out5.transpose(2, 4, 0, 1, 3).reshape(B, H, D)


# speedup vs baseline: 1.5115x; 1.0218x over previous
import functools

import jax
import jax.numpy as jnp
from jax import lax
from jax.experimental import pallas as pl
from jax.experimental.pallas import tpu as pltpu
from jax.experimental.pallas import tpu_sc as plsc


def _make_sc_kernel(V, D, B, H):
    info = plsc.get_sparse_core_info()
    NC, NS, L = info.num_cores, info.num_subcores, info.num_lanes
    NW = NC * NS
    N = B * H
    BL = B // NW
    assert D % L == 0 and B % NW == 0 and BL % L == 0 and H % 2 == 0
    n_grp = BL // L
    d_tiles = D // 8
    out_rows = N * D // 128

    mesh = plsc.VectorSubcoreMesh(core_axis_name="c", subcore_axis_name="s")

    @functools.partial(
        pl.kernel,
        mesh=mesh,
        out_type=jax.ShapeDtypeStruct((out_rows, 128), jnp.float32),
        scratch_types=[
            pltpu.VMEM((BL * H,), jnp.int32),
            pltpu.VMEM((BL,), jnp.int32),
            pltpu.VMEM((BL,), jnp.int32),
            pltpu.VMEM((BL, D), jnp.float32),
            pltpu.VMEM((BL, D), jnp.float32),
            pltpu.VMEM((D, BL), jnp.float32),
            pltpu.VMEM((D, BL), jnp.float32),
            pltpu.SemaphoreType.DMA,
            pltpu.SemaphoreType.DMA,
            pltpu.SemaphoreType.DMA,
            pltpu.SemaphoreType.DMA,
        ],
        compiler_params=pltpu.CompilerParams(
            needs_layout_passes=False, use_tc_tiling_on_sc=False
        ),
    )
    def sc_kernel(emb_hbm, idx_hbm, out_hbm, idx_v, icol_a, icol_b,
                  gbuf_a, gbuf_b, tbuf_a, tbuf_b,
                  gsem_a, gsem_b, osem_a, osem_b):
        wid = lax.axis_index("s") * NC + lax.axis_index("c")
        lane = lax.iota(jnp.int32, L)
        zero16 = jnp.zeros((L,), jnp.int32)
        mask_d = D - 1
        pltpu.sync_copy(idx_hbm.at[pl.ds(wid * BL * H, BL * H)], idx_v)

        def build_idxcol(h, icol):
            for gg in range(n_grp):
                bvec = (gg * L + lane) * H + h
                vals = plsc.load_gather(idx_v, [bvec])
                icol[pl.ds(gg * L, L)] = vals

        def fire_gather(icol, gbuf, gsem):
            return pltpu.async_copy(emb_hbm.at[icol], gbuf, gsem)

        def drain_gather(icol, gbuf, gsem):
            pltpu.make_async_copy(emb_hbm.at[icol], gbuf, gsem).wait()

        def fixup_transpose(gbuf, tbuf):
            @plsc.parallel_loop(0, n_grp)
            def grp_body(gg):
                b_ids = gg * L + lane

                def s_body(so, accs):
                    new = list(accs)
                    for si in range(16):
                        diag = (lane + (so * 16 + si)) & mask_d
                        v = plsc.load_gather(gbuf, [b_ids, diag])
                        new[si % 8] = new[si % 8] + v * v
                        plsc.store_scatter(tbuf, [diag, b_ids], v)
                    return tuple(new)

                accs = list(lax.fori_loop(
                    0, D // 16, s_body,
                    tuple(jnp.zeros((L,), jnp.float32) for _ in range(8)),
                ))
                v0 = plsc.load_gather(gbuf, [b_ids, zero16])
                while len(accs) > 1:
                    accs = [a + b for a, b in zip(accs[0::2], accs[1::2])]
                x = jnp.maximum(accs[0] - v0 * v0 + 1.0, 1.0 + 1e-12)
                i = lax.bitcast_convert_type(x, jnp.int32)
                i = 0x5F3759DF - lax.shift_right_arithmetic(i, 1)
                y = lax.bitcast_convert_type(i, jnp.float32)
                y = y * (1.5 - 0.5 * x * y * y)
                y = y * (1.5 - 0.5 * x * y * y)
                y = y * (1.5 - 0.5 * x * y * y)
                plsc.store_scatter(tbuf, [zero16, b_ids], x * y)

        def fire_writeout(h, tbuf, osem):
            return [
                pltpu.async_copy(
                    tbuf.at[pl.ds(tr * 8, 8)],
                    out_hbm.at[pl.ds(((h * d_tiles + tr) * NW + wid) * 8, 8)],
                    osem,
                )
                for tr in range(d_tiles)
            ]

        def drain_writeout(tbuf, osem):
            pltpu.make_async_copy(tbuf, out_hbm.at[pl.ds(0, D)], osem).wait()

        build_idxcol(0, icol_a)
        fire_gather(icol_a, gbuf_a, gsem_a)
        fire_writeout(0, tbuf_a, osem_a)
        fire_writeout(1, tbuf_b, osem_b)

        def round_body(q, carry):
            h0 = 2 * q
            h1 = h0 + 1
            build_idxcol(h1, icol_b)
            fire_gather(icol_b, gbuf_b, gsem_b)
            drain_gather(icol_a, gbuf_a, gsem_a)
            drain_writeout(tbuf_a, osem_a)
            fixup_transpose(gbuf_a, tbuf_a)
            fire_writeout(h0, tbuf_a, osem_a)
            h2 = jnp.minimum(h0 + 2, H - 2)
            build_idxcol(h2, icol_a)
            fire_gather(icol_a, gbuf_a, gsem_a)
            drain_gather(icol_b, gbuf_b, gsem_b)
            drain_writeout(tbuf_b, osem_b)
            fixup_transpose(gbuf_b, tbuf_b)
            fire_writeout(h1, tbuf_b, osem_b)
            return carry

        lax.fori_loop(0, H // 2, round_body, 0)
        drain_gather(icol_a, gbuf_a, gsem_a)
        drain_writeout(tbuf_a, osem_a)
        drain_writeout(tbuf_b, osem_b)

    return sc_kernel


def kernel(embeddings, indices):
    V, D = embeddings.shape
    B, H = indices.shape
    idx_flat = indices.reshape(-1).astype(jnp.int32)
    sc = _make_sc_kernel(V, D, B, H)
    out2 = sc(embeddings, idx_flat)
    nw = 32
    out5 = out2.reshape(H, D // 8, nw, 8, B // nw)
    return out5.transpose(2, 4, 0, 1, 3).reshape(B, H, D)


# speedup vs baseline: 1.5213x; 1.0065x over previous
import functools

import jax
import jax.numpy as jnp
from jax import lax
from jax.experimental import pallas as pl
from jax.experimental.pallas import tpu as pltpu
from jax.experimental.pallas import tpu_sc as plsc


def _make_sc_kernel(V, D, B, H):
    info = plsc.get_sparse_core_info()
    NC, NS, L = info.num_cores, info.num_subcores, info.num_lanes
    NW = NC * NS
    N = B * H
    BL = B // NW
    assert D % L == 0 and B % NW == 0 and BL % L == 0 and H % 2 == 0
    n_grp = BL // L
    d_tiles = D // 8
    out_rows = N * D // 128

    mesh = plsc.VectorSubcoreMesh(core_axis_name="c", subcore_axis_name="s")

    @functools.partial(
        pl.kernel,
        mesh=mesh,
        out_type=jax.ShapeDtypeStruct((out_rows, 128), jnp.float32),
        scratch_types=[
            pltpu.VMEM((BL * H,), jnp.int32),
            pltpu.VMEM((BL, D), jnp.float32),
            pltpu.VMEM((BL, D), jnp.float32),
            pltpu.VMEM((D, BL), jnp.float32),
            pltpu.VMEM((D, BL), jnp.float32),
            pltpu.SemaphoreType.DMA,
            pltpu.SemaphoreType.DMA,
            pltpu.SemaphoreType.DMA,
            pltpu.SemaphoreType.DMA,
            pltpu.SemaphoreType.DMA,
        ],
        compiler_params=pltpu.CompilerParams(
            needs_layout_passes=False, use_tc_tiling_on_sc=False
        ),
    )
    def sc_kernel(emb_hbm, idx_hbm, out_hbm, idx_v,
                  gbuf_a, gbuf_b, tbuf_a, tbuf_b,
                  isem, gsem_a, gsem_b, osem_a, osem_b):
        wid = lax.axis_index("s") * NC + lax.axis_index("c")
        lane = lax.iota(jnp.int32, L)
        zero16 = jnp.zeros((L,), jnp.int32)
        mask_d = D - 1
        for h in range(H):
            pltpu.async_copy(
                idx_hbm.at[pl.ds(h * B + wid * BL, BL)],
                idx_v.at[pl.ds(h * BL, BL)],
                isem,
            )
        pltpu.make_async_copy(
            idx_hbm.at[pl.ds(0, BL * H)], idx_v, isem
        ).wait()

        def fire_gather(h, gbuf, gsem):
            return pltpu.async_copy(
                emb_hbm.at[idx_v.at[pl.ds(h * BL, BL)]], gbuf, gsem
            )

        def drain_gather(gbuf, gsem):
            pltpu.make_async_copy(
                emb_hbm.at[idx_v.at[pl.ds(0, BL)]], gbuf, gsem
            ).wait()

        def fixup_transpose(gbuf, tbuf):
            @plsc.parallel_loop(0, n_grp)
            def grp_body(gg):
                b_ids = gg * L + lane

                def s_body(so, accs):
                    new = list(accs)
                    for si in range(16):
                        diag = (lane + (so * 16 + si)) & mask_d
                        v = plsc.load_gather(gbuf, [b_ids, diag])
                        new[si % 8] = new[si % 8] + v * v
                        plsc.store_scatter(tbuf, [diag, b_ids], v)
                    return tuple(new)

                accs = list(lax.fori_loop(
                    0, D // 16, s_body,
                    tuple(jnp.zeros((L,), jnp.float32) for _ in range(8)),
                ))
                v0 = plsc.load_gather(gbuf, [b_ids, zero16])
                while len(accs) > 1:
                    accs = [a + b for a, b in zip(accs[0::2], accs[1::2])]
                x = jnp.maximum(accs[0] - v0 * v0 + 1.0, 1.0 + 1e-12)
                i = lax.bitcast_convert_type(x, jnp.int32)
                i = 0x5F3759DF - lax.shift_right_arithmetic(i, 1)
                y = lax.bitcast_convert_type(i, jnp.float32)
                y = y * (1.5 - 0.5 * x * y * y)
                y = y * (1.5 - 0.5 * x * y * y)
                y = y * (1.5 - 0.5 * x * y * y)
                plsc.store_scatter(tbuf, [zero16, b_ids], x * y)

        def fire_writeout(h, tbuf, osem):
            return [
                pltpu.async_copy(
                    tbuf.at[pl.ds(tr * 8, 8)],
                    out_hbm.at[pl.ds(((h * d_tiles + tr) * NW + wid) * 8, 8)],
                    osem,
                )
                for tr in range(d_tiles)
            ]

        def drain_writeout(tbuf, osem):
            pltpu.make_async_copy(tbuf, out_hbm.at[pl.ds(0, D)], osem).wait()

        fire_gather(0, gbuf_a, gsem_a)
        fire_writeout(0, tbuf_a, osem_a)
        fire_writeout(1, tbuf_b, osem_b)

        def round_body(q, carry):
            h0 = 2 * q
            h1 = h0 + 1
            fire_gather(h1, gbuf_b, gsem_b)
            drain_gather(gbuf_a, gsem_a)
            drain_writeout(tbuf_a, osem_a)
            fixup_transpose(gbuf_a, tbuf_a)
            fire_writeout(h0, tbuf_a, osem_a)
            h2 = jnp.minimum(h0 + 2, H - 2)
            fire_gather(h2, gbuf_a, gsem_a)
            drain_gather(gbuf_b, gsem_b)
            drain_writeout(tbuf_b, osem_b)
            fixup_transpose(gbuf_b, tbuf_b)
            fire_writeout(h1, tbuf_b, osem_b)
            return carry

        lax.fori_loop(0, H // 2, round_body, 0)
        drain_gather(gbuf_a, gsem_a)
        drain_writeout(tbuf_a, osem_a)
        drain_writeout(tbuf_b, osem_b)

    return sc_kernel


def kernel(embeddings, indices):
    V, D = embeddings.shape
    B, H = indices.shape
    idx_cm = indices.T.reshape(-1).astype(jnp.int32)
    sc = _make_sc_kernel(V, D, B, H)
    out2 = sc(embeddings, idx_cm)
    nw = 32
    out5 = out2.reshape(H, D // 8, nw, 8, B // nw)
    return out5.transpose(2, 4, 0, 1, 3).reshape(B, H, D)
